# baseline (speedup 1.0000x reference)
import functools

import jax
import jax.numpy as jnp
from jax import lax
from jax.experimental import pallas as pl
from jax.experimental.pallas import tpu as pltpu
from jax.experimental.pallas import tpu_sc as plsc

K = 8192
D = 256
N = 4096
B = 4
RB = N // B
CB = 8192


def _argmin_body(x_ref, c_ref, idx_ref, loss_ref, acc_ref):
    i = pl.program_id(0)
    x = lax.transpose(x_ref[...], (1, 0))
    c = c_ref[...]
    xn = jnp.sum(x * x, axis=1, keepdims=True)
    cn = jnp.sum(c * c, axis=1)[None, :]
    mm2 = lax.dot_general(x * (-2.0), c, (((1,), (1,)), ((), ())),
                          preferred_element_type=jnp.float32)
    d = (xn + cn) + mm2
    m_loc = jnp.min(d, axis=1, keepdims=True)
    cols = lax.broadcasted_iota(jnp.int32, (1, CB), 1).astype(jnp.float32)
    i_loc = jnp.min(jnp.where(d == m_loc, cols, jnp.inf), axis=1, keepdims=True)
    idx_ref[...] = i_loc.astype(jnp.int32)
    s = jnp.sum(m_loc)

    @pl.when(i == 0)
    def _():
        acc_ref[0] = 0.0

    acc_ref[0] += s

    @pl.when(i == B - 1)
    def _():
        loss_ref[...] = (1.25 * (acc_ref[0] * (1.0 / (N * D)))).reshape(1, 1)


def _argmin_indices(x2, codebook):
    return pl.pallas_call(
        _argmin_body,
        grid=(B,),
        in_specs=[
            pl.BlockSpec((D, RB), lambda i: (i, 0)),
            pl.BlockSpec((CB, D), lambda i: (0, 0)),
        ],
        out_specs=[
            pl.BlockSpec((RB, 1), lambda i: (i, 0)),
            pl.BlockSpec((1, 1), lambda i: (0, 0)),
        ],
        out_shape=[
            jax.ShapeDtypeStruct((N, 1), jnp.int32),
            jax.ShapeDtypeStruct((1, 1), jnp.float32),
        ],
        scratch_shapes=[pltpu.SMEM((1,), jnp.float32)],
    )(x2, codebook)


def _make_sc_gather():
    info = plsc.get_sparse_core_info()
    nw = info.num_cores * info.num_subcores
    bpw = N // nw
    mesh = plsc.VectorSubcoreMesh(core_axis_name="c", subcore_axis_name="s")

    @functools.partial(
        pl.kernel,
        mesh=mesh,
        out_type=jax.ShapeDtypeStruct((N, D), jnp.float32),
        scratch_types=[
            pltpu.VMEM((bpw,), jnp.int32),
            pltpu.VMEM((bpw, D), jnp.float32),
            pltpu.SemaphoreType.DMA,
        ],
    )
    def gather_k(idx_hbm, table_hbm, out_hbm, idx_v, rows_v, sem):
        wid = lax.axis_index("s") * info.num_cores + lax.axis_index("c")
        base = wid * bpw
        pltpu.sync_copy(idx_hbm.at[pl.ds(base, bpw)], idx_v)
        pltpu.async_copy(table_hbm.at[idx_v], rows_v, sem).wait()
        pltpu.sync_copy(rows_v, out_hbm.at[pl.ds(base, bpw)])

    return gather_k


_sc_gather_cache = []


def _sc_gather(idx, table):
    if not _sc_gather_cache:
        _sc_gather_cache.append(_make_sc_gather())
    return _sc_gather_cache[0](idx, table)


def _transpose_body(q_ref, o_ref):
    o_ref[...] = lax.transpose(q_ref[...], (1, 0))


def _transpose_out(q):
    return pl.pallas_call(
        _transpose_body,
        grid=(B,),
        in_specs=[pl.BlockSpec((RB, D), lambda b: (b, 0))],
        out_specs=pl.BlockSpec((D, RB), lambda b: (b, 0)),
        out_shape=jax.ShapeDtypeStruct((B * D, RB), jnp.float32),
    )(q)


def kernel(inputs, codebook):
    x2 = inputs.reshape(B * D, RB)
    idx2, loss = _argmin_indices(x2, codebook)
    idx = idx2.reshape(N)
    q = _sc_gather(idx, codebook)
    quant = _transpose_out(q).reshape(inputs.shape)
    return (quant, loss.reshape(()), idx)


# speedup vs baseline: 1.2298x; 1.2298x over previous
import functools

import jax
import jax.numpy as jnp
from jax import lax
from jax.experimental import pallas as pl
from jax.experimental.pallas import tpu as pltpu
from jax.experimental.pallas import tpu_sc as plsc

K = 8192
D = 256
N = 4096
B = 4
RB = N // B
CB = 8192


def _argmin_body(x_ref, c_ref, idx_ref, loss_ref, acc_ref):
    i = pl.program_id(0)
    x = x_ref[...]
    c = c_ref[...]
    xn = jnp.sum(x * x, axis=1, keepdims=True)
    cn = jnp.sum(c * c, axis=1)[None, :]
    mm2 = lax.dot_general(x * (-2.0), c, (((1,), (1,)), ((), ())),
                          preferred_element_type=jnp.float32)
    d = (xn + cn) + mm2
    m_loc = jnp.min(d, axis=1, keepdims=True)
    cols = lax.broadcasted_iota(jnp.int32, (1, CB), 1).astype(jnp.float32)
    i_loc = jnp.min(jnp.where(d == m_loc, cols, jnp.inf), axis=1, keepdims=True)
    idx_ref[...] = i_loc.astype(jnp.int32)
    s = jnp.sum(m_loc)

    @pl.when(i == 0)
    def _():
        acc_ref[0] = 0.0

    acc_ref[0] += s

    @pl.when(i == B // 2 - 1)
    def _():
        loss_ref[...] = (1.25 * (acc_ref[0] * (1.0 / (N * D)))).reshape(1, 1)


def _argmin_indices(flat, codebook, half):
    return pl.pallas_call(
        _argmin_body,
        grid=(B // 2,),
        in_specs=[
            pl.BlockSpec((RB, D), lambda i: (i + 2 * half, 0)),
            pl.BlockSpec((CB, D), lambda i: (0, 0)),
        ],
        out_specs=[
            pl.BlockSpec((RB, 1), lambda i: (i, 0)),
            pl.BlockSpec((1, 1), lambda i: (0, 0)),
        ],
        out_shape=[
            jax.ShapeDtypeStruct((N // 2, 1), jnp.int32),
            jax.ShapeDtypeStruct((1, 1), jnp.float32),
        ],
        scratch_shapes=[pltpu.SMEM((1,), jnp.float32)],
    )(flat, codebook)


def _make_sc_gather(n):
    info = plsc.get_sparse_core_info()
    nw = info.num_cores * info.num_subcores
    bpw = n // nw
    mesh = plsc.VectorSubcoreMesh(core_axis_name="c", subcore_axis_name="s")

    @functools.partial(
        pl.kernel,
        mesh=mesh,
        out_type=jax.ShapeDtypeStruct((n, D), jnp.float32),
        scratch_types=[
            pltpu.VMEM((bpw,), jnp.int32),
            pltpu.VMEM((bpw, D), jnp.float32),
            pltpu.SemaphoreType.DMA,
        ],
    )
    def gather_k(idx_hbm, table_hbm, out_hbm, idx_v, rows_v, sem):
        wid = lax.axis_index("s") * info.num_cores + lax.axis_index("c")
        base = wid * bpw
        pltpu.sync_copy(idx_hbm.at[pl.ds(base, bpw)], idx_v)
        pltpu.async_copy(table_hbm.at[idx_v], rows_v, sem).wait()
        pltpu.sync_copy(rows_v, out_hbm.at[pl.ds(base, bpw)])

    return gather_k


_sc_gather_cache = {}


def _sc_gather(idx, table):
    n = idx.shape[0]
    if n not in _sc_gather_cache:
        _sc_gather_cache[n] = _make_sc_gather(n)
    return _sc_gather_cache[n](idx, table)


def kernel(inputs, codebook):
    x = jnp.transpose(inputs, (0, 2, 3, 1))
    flat = x.reshape(-1, D)
    idx_a, loss_a = _argmin_indices(flat, codebook, 0)
    q_a = _sc_gather(idx_a.reshape(N // 2), codebook)
    idx_b, loss_b = _argmin_indices(flat, codebook, 1)
    q_b = _sc_gather(idx_b.reshape(N // 2), codebook)
    hx = (B // 2, 32, 32, D)
    quant = jnp.concatenate([
        jnp.transpose(q_a.reshape(hx), (0, 3, 1, 2)),
        jnp.transpose(q_b.reshape(hx), (0, 3, 1, 2)),
    ], axis=0)
    idx = jnp.concatenate([idx_a, idx_b], axis=0).reshape(N)
    loss = loss_a.reshape(()) + loss_b.reshape(())
    return (quant, loss, idx)


# speedup vs baseline: 1.4552x; 1.1833x over previous
import functools

import jax
import jax.numpy as jnp
from jax import lax
from jax.experimental import pallas as pl
from jax.experimental.pallas import tpu as pltpu
from jax.experimental.pallas import tpu_sc as plsc

K = 8192
D = 256
N = 4096
B = 4
RB = N // B
CB = 8192


def _argmin_body(x_ref, c_ref, idx_ref, loss_ref, acc_ref):
    i = pl.program_id(0)
    x = x_ref[...]
    c = c_ref[...]
    xn = jnp.sum(x * x, axis=1, keepdims=True)
    cn = jnp.sum(c * c, axis=1)[None, :]
    mm2 = lax.dot_general(x * (-2.0), c, (((1,), (1,)), ((), ())),
                          preferred_element_type=jnp.float32)
    d = (xn + cn) + mm2
    m_loc = jnp.min(d, axis=1, keepdims=True)
    cols = lax.broadcasted_iota(jnp.int32, (1, CB), 1).astype(jnp.float32)
    i_loc = jnp.min(jnp.where(d == m_loc, cols, jnp.inf), axis=1, keepdims=True)
    idx_ref[...] = i_loc.astype(jnp.int32)
    s = jnp.sum(m_loc)

    @pl.when(i == 0)
    def _():
        acc_ref[0] = 0.0

    acc_ref[0] += s

    @pl.when(i == B - 1)
    def _():
        loss_ref[...] = (1.25 * (acc_ref[0] * (1.0 / (N * D)))).reshape(1, 1)


def _argmin_indices(flat, codebook):
    return pl.pallas_call(
        _argmin_body,
        grid=(B,),
        in_specs=[
            pl.BlockSpec((RB, D), lambda i: (i, 0)),
            pl.BlockSpec((CB, D), lambda i: (0, 0)),
        ],
        out_specs=[
            pl.BlockSpec((RB, 1), lambda i: (i, 0)),
            pl.BlockSpec((1, 1), lambda i: (0, 0)),
        ],
        out_shape=[
            jax.ShapeDtypeStruct((N, 1), jnp.int32),
            jax.ShapeDtypeStruct((1, 1), jnp.float32),
        ],
        scratch_shapes=[pltpu.SMEM((1,), jnp.float32)],
    )(flat, codebook)


def _make_sc_gather():
    info = plsc.get_sparse_core_info()
    nw = info.num_cores * info.num_subcores
    bpw = N // nw
    mesh = plsc.VectorSubcoreMesh(core_axis_name="c", subcore_axis_name="s")

    @functools.partial(
        pl.kernel,
        mesh=mesh,
        out_type=jax.ShapeDtypeStruct((N, D), jnp.float32),
        scratch_types=[
            pltpu.VMEM((bpw,), jnp.int32),
            pltpu.VMEM((bpw, D), jnp.float32),
            pltpu.SemaphoreType.DMA,
        ],
    )
    def gather_k(idx_hbm, table_hbm, out_hbm, idx_v, rows_v, sem):
        wid = lax.axis_index("s") * info.num_cores + lax.axis_index("c")
        base = wid * bpw
        pltpu.sync_copy(idx_hbm.at[pl.ds(base, bpw)], idx_v)
        pltpu.async_copy(table_hbm.at[idx_v], rows_v, sem).wait()
        pltpu.sync_copy(rows_v, out_hbm.at[pl.ds(base, bpw)])

    return gather_k


_sc_gather_cache = []


def _sc_gather(idx, table):
    if not _sc_gather_cache:
        _sc_gather_cache.append(_make_sc_gather())
    return _sc_gather_cache[0](idx, table)


def kernel(inputs, codebook):
    x = jnp.transpose(inputs, (0, 2, 3, 1))
    flat = x.reshape(-1, D)
    idx2, loss = _argmin_indices(flat, codebook)
    idx = idx2.reshape(N)
    q = _sc_gather(idx, codebook)
    quant = jnp.transpose(q.reshape(x.shape), (0, 3, 1, 2))
    return (quant, loss.reshape(()), idx)
